# baseline (speedup 1.0000x reference)
import jax
import jax.numpy as jnp
from jax.experimental import pallas as pl
from jax.experimental.pallas import tpu as pltpu

CAPACITY = 100000
D_MODEL = 512
BATCH = 4096

BLOCK_ROWS = 2048
NB_EMB = BATCH // BLOCK_ROWS
GRID = (CAPACITY + BLOCK_ROWS - 1) // BLOCK_ROWS


def _body(emb_ref, sig_ref, mem_ref, imp_ref, out_mem_ref, out_imp_ref):
    i = pl.program_id(0)

    @pl.when(i < NB_EMB)
    def _():
        out_mem_ref[...] = emb_ref[...]
        out_imp_ref[...] = sig_ref[...]

    @pl.when(i >= NB_EMB)
    def _():
        out_mem_ref[...] = mem_ref[...]
        out_imp_ref[...] = imp_ref[...]


def kernel(embeddings, loss_signal, memory, importance):
    emb_last = NB_EMB - 1
    out_mem, out_imp = pl.pallas_call(
        _body,
        grid=(GRID,),
        in_specs=[
            pl.BlockSpec((BLOCK_ROWS, D_MODEL), lambda i: (jnp.minimum(i, emb_last), 0)),
            pl.BlockSpec((BLOCK_ROWS,), lambda i: (jnp.minimum(i, emb_last),)),
            pl.BlockSpec((BLOCK_ROWS, D_MODEL), lambda i: (jnp.maximum(i, NB_EMB), 0)),
            pl.BlockSpec((BLOCK_ROWS,), lambda i: (jnp.maximum(i, NB_EMB),)),
        ],
        out_specs=[
            pl.BlockSpec((BLOCK_ROWS, D_MODEL), lambda i: (i, 0)),
            pl.BlockSpec((BLOCK_ROWS,), lambda i: (i,)),
        ],
        out_shape=[
            jax.ShapeDtypeStruct((CAPACITY, D_MODEL), jnp.float32),
            jax.ShapeDtypeStruct((CAPACITY,), jnp.float32),
        ],
        compiler_params=pltpu.CompilerParams(
            dimension_semantics=("parallel",)),
    )(embeddings, loss_signal, memory, importance)

    return out_mem, out_imp


# speedup vs baseline: 1.0022x; 1.0022x over previous
import jax
import jax.numpy as jnp
from jax.experimental import pallas as pl
from jax.experimental.pallas import tpu as pltpu

CAPACITY = 100000
D_MODEL = 512
BATCH = 4096

BLOCK_ROWS = 6144
GRID = (CAPACITY + BLOCK_ROWS - 1) // BLOCK_ROWS


def _body(emb_ref, sig_ref, mem_ref, imp_ref, out_mem_ref, out_imp_ref):
    i = pl.program_id(0)

    @pl.when(i == 0)
    def _():
        out_mem_ref[0:BATCH, :] = emb_ref[...]
        out_mem_ref[BATCH:BLOCK_ROWS, :] = mem_ref[BATCH:BLOCK_ROWS, :]
        out_imp_ref[0:BATCH] = sig_ref[...]
        out_imp_ref[BATCH:BLOCK_ROWS] = imp_ref[BATCH:BLOCK_ROWS]

    @pl.when(i > 0)
    def _():
        out_mem_ref[...] = mem_ref[...]
        out_imp_ref[...] = imp_ref[...]


def kernel(embeddings, loss_signal, memory, importance):
    out_mem, out_imp = pl.pallas_call(
        _body,
        grid=(GRID,),
        in_specs=[
            pl.BlockSpec((BATCH, D_MODEL), lambda i: (0, 0)),
            pl.BlockSpec((BATCH,), lambda i: (0,)),
            pl.BlockSpec((BLOCK_ROWS, D_MODEL), lambda i: (i, 0)),
            pl.BlockSpec((BLOCK_ROWS,), lambda i: (i,)),
        ],
        out_specs=[
            pl.BlockSpec((BLOCK_ROWS, D_MODEL), lambda i: (i, 0)),
            pl.BlockSpec((BLOCK_ROWS,), lambda i: (i,)),
        ],
        out_shape=[
            jax.ShapeDtypeStruct((CAPACITY, D_MODEL), jnp.float32),
            jax.ShapeDtypeStruct((CAPACITY,), jnp.float32),
        ],
        compiler_params=pltpu.CompilerParams(
            dimension_semantics=("parallel",),
            vmem_limit_bytes=63 * 1024 * 1024,
        ),
    )(embeddings, loss_signal, memory, importance)

    return out_mem, out_imp
